# Initial kernel scaffold; baseline (speedup 1.0000x reference)
import jax
import jax.numpy as jnp
from jax.experimental import pallas as pl


def kernel(numerical_input, categorical_inputs, emb_table, bot_0, bot_1, bot_2, bot_3, bot_4, bot_5, top_0, top_1, top_2, top_3, top_4, top_5, top_6, top_7, top_8, top_9):
    raise NotImplementedError("write your pallas kernel here")


# speedup vs baseline: 5.8288x; 5.8288x over previous
import functools

import jax
import jax.numpy as jnp
import numpy as np
from jax import lax
from jax.experimental import pallas as pl
from jax.experimental.pallas import tpu as pltpu
from jax.experimental.pallas import tpu_sc as plsc

B = 16384
NUM_SPARSE = 26
VOCAB = 100000
EMB_DIM = 16
N_FEAT = NUM_SPARSE + 1

_NC, _NS = 2, 16
_NW = _NC * _NS
_TOTAL_ROWS = B * NUM_SPARSE
_ROWS_PER_W = _TOTAL_ROWS // _NW
_CHUNK = 128
_CPG = 8
_GROUP = _CHUNK * _CPG
_NGROUPS = _ROWS_PER_W // _GROUP
_NCHUNKS = _ROWS_PER_W // _CHUNK


def _sc_gather_body(table_hbm, idx_hbm, out_hbm, idx_v, rows_v, sem):
    wid = lax.axis_index("s") * _NC + lax.axis_index("c")
    pltpu.sync_copy(idx_hbm.at[wid], idx_v)
    base = wid * _ROWS_PER_W

    def group(g, carry):
        handles = []
        for j in range(_CPG):
            handles.append(
                pltpu.async_copy(
                    table_hbm.at[idx_v.at[g * _CPG + j]],
                    rows_v.at[pl.ds(j * _CHUNK, _CHUNK)],
                    sem,
                )
            )
        for h in handles:
            h.wait()
        start = pl.multiple_of(base + g * _GROUP, _GROUP)
        pltpu.sync_copy(rows_v, out_hbm.at[pl.ds(start, _GROUP)])
        return carry

    lax.fori_loop(0, _NGROUPS, group, 0)


@functools.cache
def _sc_gather():
    return pl.kernel(
        _sc_gather_body,
        mesh=plsc.VectorSubcoreMesh(core_axis_name="c", subcore_axis_name="s"),
        out_type=jax.ShapeDtypeStruct((_TOTAL_ROWS, EMB_DIM), jnp.float32),
        scratch_types=[
            pltpu.VMEM((_NCHUNKS, _CHUNK), jnp.int32),
            pltpu.VMEM((_GROUP, EMB_DIM), jnp.float32),
            pltpu.SemaphoreType.DMA,
        ],
        compiler_params=pltpu.CompilerParams(use_tc_tiling_on_sc=False),
    )


_BB = 512


def _tc_dlrm_body(num_ref, emb_ref, b0w, b0b, b1w, b1b, b2w, b2b,
                  w0a, w0z, t0b, t1w, t1b, t2w, t2b, t3w, t3b, t4w, t4b,
                  out_ref):
    f32 = jnp.float32
    x = num_ref[...]
    h = jax.nn.relu(jnp.dot(x, b0w[...], preferred_element_type=f32) + b0b[...])
    h = jax.nn.relu(jnp.dot(h, b1w[...], preferred_element_type=f32) + b1b[...])
    bot = jax.nn.relu(jnp.dot(h, b2w[...], preferred_element_type=f32) + b2b[...])

    emb = emb_ref[...].reshape(_BB, NUM_SPARSE, EMB_DIM)
    xcat = jnp.concatenate([bot.reshape(_BB, 1, EMB_DIM), emb], axis=1)
    z = lax.dot_general(xcat, xcat, (((2,), (2,)), ((0,), (0,))),
                        preferred_element_type=f32)
    zflat = z.reshape(_BB, N_FEAT * N_FEAT)

    t = jnp.dot(bot, w0a[...], preferred_element_type=f32)
    t = t + jnp.dot(zflat, w0z[...], preferred_element_type=f32)
    t = jax.nn.relu(t + t0b[...])
    t = jax.nn.relu(jnp.dot(t, t1w[...], preferred_element_type=f32) + t1b[...])
    t = jax.nn.relu(jnp.dot(t, t2w[...], preferred_element_type=f32) + t2b[...])
    t = jax.nn.relu(jnp.dot(t, t3w[...], preferred_element_type=f32) + t3b[...])
    out_ref[...] = jnp.dot(t, t4w[...], preferred_element_type=f32) + t4b[...]


def _tc_dlrm(num, emb2d, b0w, b0b, b1w, b1b, b2w, b2b,
             w0a, w0z, t0b, t1w, t1b, t2w, t2b, t3w, t3b, t4w, t4b):
    grid = (B // _BB,)
    full = lambda a: pl.BlockSpec(a.shape, lambda i: (0,) * a.ndim)
    in_specs = [
        pl.BlockSpec((_BB, num.shape[1]), lambda i: (i, 0)),
        pl.BlockSpec((_BB, emb2d.shape[1]), lambda i: (i, 0)),
    ] + [full(a) for a in (b0w, b0b, b1w, b1b, b2w, b2b,
                           w0a, w0z, t0b, t1w, t1b, t2w, t2b, t3w, t3b,
                           t4w, t4b)]
    return pl.pallas_call(
        _tc_dlrm_body,
        grid=grid,
        in_specs=in_specs,
        out_specs=pl.BlockSpec((_BB, 1), lambda i: (i, 0)),
        out_shape=jax.ShapeDtypeStruct((B, 1), jnp.float32),
    )(num, emb2d, b0w, b0b, b1w, b1b, b2w, b2b,
      w0a, w0z, t0b, t1w, t1b, t2w, t2b, t3w, t3b, t4w, t4b)


_LI, _LJ = np.tril_indices(N_FEAT, -1)
_PAIR_POS = np.asarray(_LI * N_FEAT + _LJ, dtype=np.int32)


def kernel(numerical_input, categorical_inputs, emb_table,
           bot_0, bot_1, bot_2, bot_3, bot_4, bot_5,
           top_0, top_1, top_2, top_3, top_4,
           top_5, top_6, top_7, top_8, top_9):
    offsets = jnp.arange(NUM_SPARSE, dtype=categorical_inputs.dtype) * VOCAB
    idx = (categorical_inputs + offsets[None, :]).reshape(_NW, _NCHUNKS, _CHUNK)
    emb_flat = _sc_gather()(emb_table, idx)
    emb2d = emb_flat.reshape(B, NUM_SPARSE * EMB_DIM)

    w0a = top_0[:EMB_DIM]
    w0z = jnp.zeros((N_FEAT * N_FEAT, top_0.shape[1]), top_0.dtype)
    w0z = w0z.at[_PAIR_POS].set(top_0[EMB_DIM:])

    row = lambda b: b.reshape(1, -1)
    return _tc_dlrm(numerical_input, emb2d,
                    bot_0, row(bot_1), bot_2, row(bot_3), bot_4, row(bot_5),
                    w0a, w0z, row(top_1), top_2, row(top_3), top_4, row(top_5),
                    top_6, row(top_7), top_8, row(top_9))
